# baseline (device time: 19257 ns/iter reference)
import jax
import jax.numpy as jnp
from jax import lax
from jax.experimental import pallas as pl
from jax.experimental.pallas import tpu as pltpu

Z = 4
X = 2


def kernel(partial, resid, gamma):
    _, m, d = partial.shape
    mo = m // (Z * X)

    def body(p_ref, r_ref, g_ref, out_ref,
             my_bf, rs_recv, ag_send, agz_recv, agx_recv,
             rs_send_sems, rs_recv_sems,
             agz_send_sems, agz_recv_sems,
             agx_send_sems, agx_recv_sems):
        my_x = lax.axis_index("x")
        my_y = lax.axis_index("y")
        my_z = lax.axis_index("z")
        xp = 1 - my_x

        barrier_sem = pltpu.get_barrier_semaphore()
        for q in range(Z):
            pl.semaphore_signal(
                barrier_sem, inc=1,
                device_id=(xp, my_y, q),
                device_id_type=pl.DeviceIdType.MESH,
            )

        for q in range(Z):
            my_bf[q] = p_ref[0, pl.ds((q * X + my_x) * mo, mo), :].astype(
                jnp.bfloat16
            )

        sends = []
        for o in range(1, Z):
            peer = (my_z + o) % Z
            rdma = pltpu.make_async_remote_copy(
                src_ref=my_bf.at[peer],
                dst_ref=rs_recv.at[my_z],
                send_sem=rs_send_sems.at[o],
                recv_sem=rs_recv_sems.at[my_z],
                device_id=(my_x, my_y, peer),
                device_id_type=pl.DeviceIdType.MESH,
            )
            rdma.start()
            sends.append(rdma)

        acc = my_bf[my_z].astype(jnp.float32)
        for o in range(1, Z):
            peer = (my_z + o) % Z
            recv = pltpu.make_async_remote_copy(
                src_ref=my_bf.at[peer],
                dst_ref=rs_recv.at[peer],
                send_sem=rs_send_sems.at[o],
                recv_sem=rs_recv_sems.at[peer],
                device_id=(my_x, my_y, peer),
                device_id_type=pl.DeviceIdType.MESH,
            )
            recv.wait_recv()
            acc = acc + rs_recv[peer].astype(jnp.float32)

        row0 = (my_z * X + my_x) * mo
        y = acc + r_ref[pl.ds(row0, mo), :]
        rms = jnp.sqrt(jnp.mean(y * y, axis=-1, keepdims=True) + 1e-6)
        mine = y / rms * g_ref[...].reshape(1, d)
        ag_send[...] = mine.astype(jnp.bfloat16)

        for o in range(1, Z):
            peer = (my_z + o) % Z
            rdma = pltpu.make_async_remote_copy(
                src_ref=ag_send,
                dst_ref=agz_recv.at[my_z],
                send_sem=agz_send_sems.at[o],
                recv_sem=agz_recv_sems.at[my_z],
                device_id=(my_x, my_y, peer),
                device_id_type=pl.DeviceIdType.MESH,
            )
            rdma.start()
            sends.append(rdma)
        pl.semaphore_wait(barrier_sem, Z)
        for q in range(Z):
            rdma = pltpu.make_async_remote_copy(
                src_ref=ag_send,
                dst_ref=agx_recv.at[my_z],
                send_sem=agx_send_sems.at[q],
                recv_sem=agx_recv_sems.at[my_z],
                device_id=(xp, my_y, q),
                device_id_type=pl.DeviceIdType.MESH,
            )
            rdma.start()
            sends.append(rdma)

        out_ref[pl.ds(row0, mo), :] = mine

        for o in range(1, Z):
            peer = (my_z + o) % Z
            recv = pltpu.make_async_remote_copy(
                src_ref=ag_send,
                dst_ref=agz_recv.at[peer],
                send_sem=agz_send_sems.at[o],
                recv_sem=agz_recv_sems.at[peer],
                device_id=(my_x, my_y, peer),
                device_id_type=pl.DeviceIdType.MESH,
            )
            recv.wait_recv()
            out_ref[pl.ds((peer * X + my_x) * mo, mo), :] = (
                agz_recv[peer].astype(jnp.float32)
            )

        for q in range(Z):
            recv = pltpu.make_async_remote_copy(
                src_ref=ag_send,
                dst_ref=agx_recv.at[q],
                send_sem=agx_send_sems.at[q],
                recv_sem=agx_recv_sems.at[q],
                device_id=(xp, my_y, my_z),
                device_id_type=pl.DeviceIdType.MESH,
            )
            recv.wait_recv()
            out_ref[pl.ds((q * X + xp) * mo, mo), :] = (
                agx_recv[q].astype(jnp.float32)
            )

        for rdma in sends:
            rdma.wait_send()

    return pl.pallas_call(
        body,
        out_shape=jax.ShapeDtypeStruct((m, d), jnp.float32),
        in_specs=[
            pl.BlockSpec(memory_space=pltpu.VMEM),
            pl.BlockSpec(memory_space=pltpu.VMEM),
            pl.BlockSpec(memory_space=pltpu.VMEM),
        ],
        out_specs=pl.BlockSpec(memory_space=pltpu.VMEM),
        scratch_shapes=[
            pltpu.VMEM((Z, mo, d), jnp.bfloat16),
            pltpu.VMEM((Z, mo, d), jnp.bfloat16),
            pltpu.VMEM((mo, d), jnp.bfloat16),
            pltpu.VMEM((Z, mo, d), jnp.bfloat16),
            pltpu.VMEM((Z, mo, d), jnp.bfloat16),
            pltpu.SemaphoreType.DMA((Z,)),
            pltpu.SemaphoreType.DMA((Z,)),
            pltpu.SemaphoreType.DMA((Z,)),
            pltpu.SemaphoreType.DMA((Z,)),
            pltpu.SemaphoreType.DMA((Z,)),
            pltpu.SemaphoreType.DMA((Z,)),
        ],
        compiler_params=pltpu.CompilerParams(collective_id=0),
    )(partial, resid, gamma)
